# baseline (speedup 1.0000x reference)
import functools

import jax
import jax.numpy as jnp
from jax import lax
from jax.experimental import pallas as pl
from jax.experimental.pallas import tpu as pltpu
from jax.experimental.pallas import tpu_sc as plsc

MEM = 50000
B = 1024
D = 3072
L = 16
NW = 32
FPT = D // NW
NPAIR = FPT // 2


def _w_body(idx_col_ref, idx_row_ref, w_ref):
    eq = idx_col_ref[...] == idx_row_ref[...]
    jj = lax.broadcasted_iota(jnp.int32, (B, B), 1)
    w_ref[...] = jnp.max(jnp.where(eq, jj, -1), axis=1, keepdims=True)


def _tc_w(idx):
    return pl.pallas_call(
        _w_body,
        out_shape=jax.ShapeDtypeStruct((B, 1), jnp.int32),
    )(idx.reshape(B, 1), idx.reshape(1, B))


@functools.cache
def _mesh():
    return plsc.VectorSubcoreMesh(core_axis_name="c", subcore_axis_name="s")


def _sc_fused_body(bufT_hbm, xT_hbm, idx_hbm, w_hbm, ridx_hbm,
                   newT_hbm, rxT_hbm,
                   vrow0, vrow1, xrow0, xrow1, orow0, orow1,
                   idx_v, w_v, ridx_v,
                   insem, xinsem, onsem, oxsem):
    vrow = (vrow0, vrow1)
    xrow = (xrow0, xrow1)
    orow = (orow0, orow1)
    c = lax.axis_index("c")
    s = lax.axis_index("s")
    wid = s * 2 + c
    fbase = wid * FPT

    pltpu.sync_copy(idx_hbm, idx_v)
    pltpu.sync_copy(w_hbm, w_v)
    pltpu.sync_copy(ridx_hbm, ridx_v)

    def in_dmas(f, b):
        return (
            pltpu.make_async_copy(bufT_hbm.at[pl.ds(f * MEM, MEM)],
                                  vrow[b], insem.at[b]),
            pltpu.make_async_copy(xT_hbm.at[pl.ds(f * B, B)],
                                  xrow[b], xinsem.at[b]),
        )

    def out_dmas(f, b):
        return (
            pltpu.make_async_copy(vrow[b], newT_hbm.at[pl.ds(f * MEM, MEM)],
                                  onsem.at[b]),
            pltpu.make_async_copy(orow[b], rxT_hbm.at[pl.ds(f * B, B)],
                                  oxsem.at[b]),
        )

    for b in range(2):
        for d in in_dmas(fbase + b, b):
            d.start()

    def process(f, b, prefetch):
        for d in in_dmas(f, b):
            d.wait()
        row = vrow[b]
        xr = xrow[b]
        outr = orow[b]
        for k in range(B // L):
            sl = pl.ds(k * L, L)
            plsc.store_scatter(row, [idx_v[sl]],
                               plsc.load_gather(xr, [w_v[sl]]))
        new_dma, rx_dma = out_dmas(f, b)
        new_dma.start()
        for k in range(B // L):
            sl = pl.ds(k * L, L)
            outr[sl] = plsc.load_gather(row, [ridx_v[sl]])
        rx_dma.start()
        new_dma.wait()
        rx_dma.wait()
        if prefetch:
            for d in in_dmas(f + 2, b):
                d.start()

    def pair(rr, carry):
        f0 = fbase + rr * 2
        process(f0, 0, True)
        process(f0 + 1, 1, True)
        return carry

    lax.fori_loop(0, NPAIR - 1, pair, jnp.int32(0))
    process(fbase + FPT - 2, 0, False)
    process(fbase + FPT - 1, 1, False)


@functools.cache
def _sc_fused():
    return pl.kernel(
        _sc_fused_body,
        out_type=(
            jax.ShapeDtypeStruct((D * MEM,), jnp.float32),
            jax.ShapeDtypeStruct((D * B,), jnp.float32),
        ),
        mesh=_mesh(),
        compiler_params=pltpu.CompilerParams(needs_layout_passes=False),
        scratch_types=[
            pltpu.VMEM((MEM,), jnp.float32),
            pltpu.VMEM((MEM,), jnp.float32),
            pltpu.VMEM((B,), jnp.float32),
            pltpu.VMEM((B,), jnp.float32),
            pltpu.VMEM((B,), jnp.float32),
            pltpu.VMEM((B,), jnp.float32),
            pltpu.VMEM((B,), jnp.int32),
            pltpu.VMEM((B,), jnp.int32),
            pltpu.VMEM((B,), jnp.int32),
            pltpu.SemaphoreType.DMA((2,)),
            pltpu.SemaphoreType.DMA((2,)),
            pltpu.SemaphoreType.DMA((2,)),
            pltpu.SemaphoreType.DMA((2,)),
        ],
    )


def _sc_label_body(lbl_hbm, y_hbm, idx_hbm, w_hbm, ridx_hbm,
                   out_lbl_hbm, out_y_hbm,
                   lblbuf_v, if_v, wf_v, yf_v, ry_v):
    c = lax.axis_index("c")
    s = lax.axis_index("s")
    wid = s * 2 + c

    @pl.when(wid == 0)
    def _():
        pltpu.sync_copy(lbl_hbm, lblbuf_v)
        pltpu.sync_copy(idx_hbm, if_v)
        pltpu.sync_copy(w_hbm, wf_v)
        pltpu.sync_copy(y_hbm, yf_v)

        def sbody(k, carry):
            sl = pl.ds(k * L, L)
            plsc.store_scatter(lblbuf_v, [if_v[sl]],
                               plsc.load_gather(yf_v, [wf_v[sl]]))
            return carry

        lax.fori_loop(0, B // L, sbody, 0)
        pltpu.sync_copy(lblbuf_v, out_lbl_hbm)

        pltpu.sync_copy(ridx_hbm, if_v)

        def gbody(k, carry):
            sl = pl.ds(k * L, L)
            ry_v[sl] = plsc.load_gather(lblbuf_v, [if_v[sl]])
            return carry

        lax.fori_loop(0, B // L, gbody, 0)
        pltpu.sync_copy(ry_v, out_y_hbm)


@functools.cache
def _sc_labels():
    return pl.kernel(
        _sc_label_body,
        out_type=(
            jax.ShapeDtypeStruct((MEM,), jnp.int32),
            jax.ShapeDtypeStruct((B,), jnp.int32),
        ),
        mesh=_mesh(),
        compiler_params=pltpu.CompilerParams(needs_layout_passes=False),
        scratch_types=[
            pltpu.VMEM((MEM,), jnp.int32),
            pltpu.VMEM((B,), jnp.int32),
            pltpu.VMEM((B,), jnp.int32),
            pltpu.VMEM((B,), jnp.int32),
            pltpu.VMEM((B,), jnp.int32),
        ],
    )


def kernel(x, buffer_img, y, buffer_label, idx, retrieve_idx):
    idx = idx.astype(jnp.int32)
    ridx = retrieve_idx.astype(jnp.int32)
    y = y.astype(jnp.int32)

    bufT = jnp.transpose(buffer_img, (1, 2, 3, 0)).reshape(D * MEM)
    xT = jnp.transpose(x, (1, 2, 3, 0)).reshape(D * B)

    w = _tc_w(idx).reshape(B)
    newT, rxT = _sc_fused()(bufT, xT, idx, w, ridx)
    new_label, ry = _sc_labels()(buffer_label, y, idx, w, ridx)

    new_buf = jnp.transpose(newT.reshape(3, 32, 32, MEM), (3, 0, 1, 2))
    rx = jnp.transpose(rxT.reshape(3, 32, 32, B), (3, 0, 1, 2))
    return (rx, ry, new_buf, new_label)


# speedup vs baseline: 4.3823x; 4.3823x over previous
import functools

import jax
import jax.numpy as jnp
from jax import lax
from jax.experimental import pallas as pl
from jax.experimental.pallas import tpu as pltpu
from jax.experimental.pallas import tpu_sc as plsc

MEM = 50000
B = 1024
D = 3072
L = 16
NW = 32
FPT = D // NW
NPAIR = FPT // 2


def _w_body(idx_col_ref, idx_row_ref, w_ref):
    eq = idx_col_ref[...] == idx_row_ref[...]
    jj = lax.broadcasted_iota(jnp.int32, (B, B), 1)
    w_ref[...] = jnp.max(jnp.where(eq, jj, -1), axis=1, keepdims=True)


def _tc_w(idx):
    return pl.pallas_call(
        _w_body,
        out_shape=jax.ShapeDtypeStruct((B, 1), jnp.int32),
    )(idx.reshape(B, 1), idx.reshape(1, B))


@functools.cache
def _mesh():
    return plsc.VectorSubcoreMesh(core_axis_name="c", subcore_axis_name="s")


def _sc_fused_body(bufT_hbm, xT_hbm, idx_hbm, w_hbm, ridx_hbm,
                   newT_hbm, rxT_hbm,
                   vrow0, vrow1, xrow0, xrow1, orow0, orow1,
                   idx_v, w_v, ridx_v,
                   insem, xinsem, onsem, oxsem):
    vrow = (vrow0, vrow1)
    xrow = (xrow0, xrow1)
    orow = (orow0, orow1)
    c = lax.axis_index("c")
    s = lax.axis_index("s")
    wid = s * 2 + c
    fbase = wid * FPT

    pltpu.sync_copy(idx_hbm, idx_v)
    pltpu.sync_copy(w_hbm, w_v)
    pltpu.sync_copy(ridx_hbm, ridx_v)

    def in_dmas(f, b):
        return (
            pltpu.make_async_copy(bufT_hbm.at[pl.ds(f, 1)],
                                  vrow[b], insem.at[b]),
            pltpu.make_async_copy(xT_hbm.at[pl.ds(f, 1)],
                                  xrow[b], xinsem.at[b]),
        )

    def out_dmas(f, b):
        return (
            pltpu.make_async_copy(vrow[b], newT_hbm.at[pl.ds(f, 1)],
                                  onsem.at[b]),
            pltpu.make_async_copy(orow[b], rxT_hbm.at[pl.ds(f, 1)],
                                  oxsem.at[b]),
        )

    for b in range(2):
        for d in in_dmas(fbase + b, b):
            d.start()

    def process(f, b, prefetch):
        for d in in_dmas(f, b):
            d.wait()
        row = vrow[b]
        xr = xrow[b]
        outr = orow[b]
        zero = jnp.zeros((L,), jnp.int32)
        for k in range(B // L):
            sl = pl.ds(k * L, L)
            plsc.store_scatter(row, [zero, idx_v[sl]],
                               plsc.load_gather(xr, [zero, w_v[sl]]))
        new_dma, rx_dma = out_dmas(f, b)
        new_dma.start()
        lanes = lax.iota(jnp.int32, L)
        for k in range(B // L):
            sl = pl.ds(k * L, L)
            plsc.store_scatter(outr, [zero, lanes + (k * L)],
                               plsc.load_gather(row, [zero, ridx_v[sl]]))
        rx_dma.start()
        new_dma.wait()
        rx_dma.wait()
        if prefetch:
            for d in in_dmas(f + 2, b):
                d.start()

    def pair(rr, carry):
        f0 = fbase + rr * 2
        process(f0, 0, True)
        process(f0 + 1, 1, True)
        return carry

    lax.fori_loop(0, NPAIR - 1, pair, jnp.int32(0))
    process(fbase + FPT - 2, 0, False)
    process(fbase + FPT - 1, 1, False)


@functools.cache
def _sc_fused():
    return pl.kernel(
        _sc_fused_body,
        out_type=(
            jax.ShapeDtypeStruct((D, MEM), jnp.float32),
            jax.ShapeDtypeStruct((D, B), jnp.float32),
        ),
        mesh=_mesh(),
        compiler_params=pltpu.CompilerParams(needs_layout_passes=False),
        scratch_types=[
            pltpu.VMEM((1, MEM), jnp.float32),
            pltpu.VMEM((1, MEM), jnp.float32),
            pltpu.VMEM((1, B), jnp.float32),
            pltpu.VMEM((1, B), jnp.float32),
            pltpu.VMEM((1, B), jnp.float32),
            pltpu.VMEM((1, B), jnp.float32),
            pltpu.VMEM((B,), jnp.int32),
            pltpu.VMEM((B,), jnp.int32),
            pltpu.VMEM((B,), jnp.int32),
            pltpu.SemaphoreType.DMA((2,)),
            pltpu.SemaphoreType.DMA((2,)),
            pltpu.SemaphoreType.DMA((2,)),
            pltpu.SemaphoreType.DMA((2,)),
        ],
    )


def _sc_label_body(lbl_hbm, y_hbm, idx_hbm, w_hbm, ridx_hbm,
                   out_lbl_hbm, out_y_hbm,
                   lblbuf_v, if_v, wf_v, yf_v, ry_v):
    c = lax.axis_index("c")
    s = lax.axis_index("s")
    wid = s * 2 + c

    @pl.when(wid == 0)
    def _():
        pltpu.sync_copy(lbl_hbm, lblbuf_v)
        pltpu.sync_copy(idx_hbm, if_v)
        pltpu.sync_copy(w_hbm, wf_v)
        pltpu.sync_copy(y_hbm, yf_v)

        def sbody(k, carry):
            sl = pl.ds(k * L, L)
            plsc.store_scatter(lblbuf_v, [if_v[sl]],
                               plsc.load_gather(yf_v, [wf_v[sl]]))
            return carry

        lax.fori_loop(0, B // L, sbody, 0)
        pltpu.sync_copy(lblbuf_v, out_lbl_hbm)

        pltpu.sync_copy(ridx_hbm, if_v)

        def gbody(k, carry):
            sl = pl.ds(k * L, L)
            ry_v[sl] = plsc.load_gather(lblbuf_v, [if_v[sl]])
            return carry

        lax.fori_loop(0, B // L, gbody, 0)
        pltpu.sync_copy(ry_v, out_y_hbm)


@functools.cache
def _sc_labels():
    return pl.kernel(
        _sc_label_body,
        out_type=(
            jax.ShapeDtypeStruct((MEM,), jnp.int32),
            jax.ShapeDtypeStruct((B,), jnp.int32),
        ),
        mesh=_mesh(),
        compiler_params=pltpu.CompilerParams(needs_layout_passes=False),
        scratch_types=[
            pltpu.VMEM((MEM,), jnp.int32),
            pltpu.VMEM((B,), jnp.int32),
            pltpu.VMEM((B,), jnp.int32),
            pltpu.VMEM((B,), jnp.int32),
            pltpu.VMEM((B,), jnp.int32),
        ],
    )


def kernel(x, buffer_img, y, buffer_label, idx, retrieve_idx):
    idx = idx.astype(jnp.int32)
    ridx = retrieve_idx.astype(jnp.int32)
    y = y.astype(jnp.int32)

    bufT = jnp.transpose(buffer_img, (1, 2, 3, 0)).reshape(D, MEM)
    xT = jnp.transpose(x, (1, 2, 3, 0)).reshape(D, B)

    w = _tc_w(idx).reshape(B)
    newT, rxT = _sc_fused()(bufT, xT, idx, w, ridx)
    new_label, ry = _sc_labels()(buffer_label, y, idx, w, ridx)

    new_buf = jnp.transpose(newT.reshape(3, 32, 32, MEM), (3, 0, 1, 2))
    rx = jnp.transpose(rxT.reshape(3, 32, 32, B), (3, 0, 1, 2))
    return (rx, ry, new_buf, new_label)
